# Initial kernel scaffold; baseline (speedup 1.0000x reference)
import jax
import jax.numpy as jnp
from jax.experimental import pallas as pl


def kernel(x, edge_index, batch_idx, focal_points, emb, W1, b1, W2, b2, W_ih, W_hh, b_ih, b_hh, W_fc1, b_fc1, W_fc2, b_fc2):
    raise NotImplementedError("write your pallas kernel here")


# speedup vs baseline: 25.4584x; 25.4584x over previous
import functools

import jax
import jax.numpy as jnp
from jax import lax
from jax.experimental import pallas as pl
from jax.experimental.pallas import tpu as pltpu
from jax.experimental.pallas import tpu_sc as plsc

N = 10000
E = 320000
H = 128
BGRAPH = 16
NC = 2
NS = 16
NW = NC * NS
NPAD = 10240
ROWS_PER_TILE = NPAD // NS
ECHUNK = 80
N_ECHUNK = E // ECHUNK
GCHUNK = 80
N_GCHUNK = N // GCHUNK

_f32 = jnp.float32

_sc_mesh = plsc.VectorSubcoreMesh(core_axis_name="c", subcore_axis_name="s")


@functools.partial(
    pl.kernel,
    out_type=[
        jax.ShapeDtypeStruct((N, H), _f32),
        jax.ShapeDtypeStruct((NC, NPAD, H), _f32),
    ],
    mesh=_sc_mesh,
    scratch_types=[
        pltpu.VMEM((GCHUNK,), jnp.int32),
        pltpu.VMEM((GCHUNK, H), _f32),
        pltpu.VMEM((ECHUNK,), jnp.int32),
        pltpu.VMEM((ECHUNK, H), _f32),
        pltpu.VMEM_SHARED((NPAD, H), _f32),
        pltpu.SemaphoreType.DMA,
    ],
)
def _embed_deg_kernel(x_hbm, ecol_hbm, emb_hbm, ones_hbm, zeros_hbm,
                      h0_hbm, degp_hbm,
                      idx_v, rows_v, col_v, ones_v, deg_sh, sem):
    c = lax.axis_index("c")
    s = lax.axis_index("s")
    w = s * NC + c

    pltpu.sync_copy(zeros_hbm,
                    deg_sh.at[pl.ds(s * ROWS_PER_TILE, ROWS_PER_TILE)])
    pltpu.sync_copy(ones_hbm, ones_v)
    plsc.subcore_barrier()

    def gbody(i, carry):
        ch = w + NW * i

        @pl.when(ch < N_GCHUNK)
        def _():
            pltpu.sync_copy(x_hbm.at[pl.ds(ch * GCHUNK, GCHUNK)], idx_v)
            pltpu.async_copy(emb_hbm.at[idx_v], rows_v, sem).wait()
            pltpu.sync_copy(rows_v, h0_hbm.at[pl.ds(ch * GCHUNK, GCHUNK)])

        return carry

    lax.fori_loop(0, (N_GCHUNK + NW - 1) // NW, gbody, 0)

    def dbody(i, carry):
        ch = w + NW * i
        pltpu.sync_copy(ecol_hbm.at[pl.ds(ch * ECHUNK, ECHUNK)], col_v)
        pltpu.sync_copy(ones_v, deg_sh.at[col_v], add=True)
        return carry

    lax.fori_loop(0, N_ECHUNK // NW, dbody, 0)
    plsc.subcore_barrier()

    pltpu.sync_copy(deg_sh.at[pl.ds(s * ROWS_PER_TILE, ROWS_PER_TILE)],
                    degp_hbm.at[c, pl.ds(s * ROWS_PER_TILE, ROWS_PER_TILE)])


@functools.partial(
    pl.kernel,
    out_type=jax.ShapeDtypeStruct((NC, NPAD, H), _f32),
    mesh=_sc_mesh,
    scratch_types=[
        pltpu.VMEM((ECHUNK,), jnp.int32),
        pltpu.VMEM((ECHUNK,), jnp.int32),
        pltpu.VMEM((ECHUNK, H), _f32),
        pltpu.VMEM_SHARED((NPAD, H), _f32),
        pltpu.SemaphoreType.DMA,
    ],
)
def _msg_kernel(erow_hbm, ecol_hbm, y_hbm, zeros_hbm, out_hbm,
                row_v, col_v, msg_v, acc_sh, sem):
    c = lax.axis_index("c")
    s = lax.axis_index("s")
    w = s * NC + c

    pltpu.sync_copy(zeros_hbm,
                    acc_sh.at[pl.ds(s * ROWS_PER_TILE, ROWS_PER_TILE)])
    plsc.subcore_barrier()

    def body(i, carry):
        ch = w + NW * i
        base = ch * ECHUNK
        pltpu.sync_copy(erow_hbm.at[pl.ds(base, ECHUNK)], row_v)
        pltpu.sync_copy(ecol_hbm.at[pl.ds(base, ECHUNK)], col_v)
        pltpu.async_copy(y_hbm.at[row_v], msg_v, sem).wait()
        pltpu.sync_copy(msg_v, acc_sh.at[col_v], add=True)
        return carry

    lax.fori_loop(0, N_ECHUNK // NW, body, 0)
    plsc.subcore_barrier()

    pltpu.sync_copy(acc_sh.at[pl.ds(s * ROWS_PER_TILE, ROWS_PER_TILE)],
                    out_hbm.at[c, pl.ds(s * ROWS_PER_TILE, ROWS_PER_TILE)])


RB = 2000
NRB = N // RB


def _proj1_body(h0_ref, w1_ref, degp_ref, y_ref, dis_ref):
    deg = degp_ref[0, :, 0:1] + degp_ref[1, :, 0:1] + 1.0
    dis = lax.rsqrt(deg)
    xw = jnp.dot(h0_ref[...], w1_ref[...], preferred_element_type=_f32,
                 precision=lax.Precision.HIGHEST)
    y_ref[...] = xw * dis
    dis_ref[...] = dis


def _proj1(h0, W1, degp):
    return pl.pallas_call(
        _proj1_body,
        grid=(NRB,),
        in_specs=[
            pl.BlockSpec((RB, H), lambda i: (i, 0)),
            pl.BlockSpec((H, H), lambda i: (0, 0)),
            pl.BlockSpec((NC, RB, H), lambda i: (0, i, 0)),
        ],
        out_specs=[
            pl.BlockSpec((RB, H), lambda i: (i, 0)),
            pl.BlockSpec((RB, 1), lambda i: (i, 0)),
        ],
        out_shape=[
            jax.ShapeDtypeStruct((N, H), _f32),
            jax.ShapeDtypeStruct((N, 1), _f32),
        ],
    )(h0, W1, degp)


def _finish_proj_body(p_ref, y_ref, dis_ref, b_ref, w_ref, out_ref,
                      *, post_scale):
    dis = dis_ref[...]
    h = jnp.maximum(dis * (p_ref[0] + p_ref[1] + y_ref[...]) + b_ref[...],
                    0.0)
    xw = jnp.dot(h, w_ref[...], preferred_element_type=_f32,
                 precision=lax.Precision.HIGHEST)
    out_ref[...] = xw * dis if post_scale else xw


def _finish_proj(p, y, dis, br, W, post_scale):
    ho = W.shape[1]
    body = functools.partial(_finish_proj_body, post_scale=post_scale)
    return pl.pallas_call(
        body,
        grid=(NRB,),
        in_specs=[
            pl.BlockSpec((NC, RB, H), lambda i: (0, i, 0)),
            pl.BlockSpec((RB, H), lambda i: (i, 0)),
            pl.BlockSpec((RB, 1), lambda i: (i, 0)),
            pl.BlockSpec((1, H), lambda i: (0, 0)),
            pl.BlockSpec((H, ho), lambda i: (0, 0)),
        ],
        out_specs=pl.BlockSpec((RB, ho), lambda i: (i, 0)),
        out_shape=jax.ShapeDtypeStruct((N, ho), _f32),
    )(p, y, dis, br, W)


def _gru_body(gx_ref, whh_ref, bih_ref, bhh_ref, offs_ref, cnts_ref, ml_ref,
              focal_ref, wfc1a_ref, wfc1b_ref, bfc1_ref, wfc2_ref, bfc2_ref,
              out_ref):
    max_len = ml_ref[0]
    bih = bih_ref[...]
    bhh = bhh_ref[...]
    whh = whh_ref[...]

    def step(t, carry):
        hst, acc = carry
        rows = []
        for b in range(BGRAPH):
            valid = t < cnts_ref[b]
            safe = jnp.where(valid, offs_ref[b] + t, 0)
            r = gx_ref[pl.ds(safe, 1), :]
            rows.append(jnp.where(valid, r, 0.0))
        gx = jnp.concatenate(rows, axis=0) + bih
        gh = jnp.dot(hst, whh, preferred_element_type=_f32,
                     precision=lax.Precision.HIGHEST) + bhh
        r_ = jax.nn.sigmoid(gx[:, :H] + gh[:, :H])
        z = jax.nn.sigmoid(gx[:, H:2 * H] + gh[:, H:2 * H])
        n_ = jnp.tanh(gx[:, 2 * H:] + r_ * gh[:, 2 * H:])
        hst = (1.0 - z) * n_ + z * hst
        return hst, acc + hst

    init = (jnp.zeros((BGRAPH, H), _f32), jnp.zeros((BGRAPH, H), _f32))
    _, acc = lax.fori_loop(0, max_len, step, init)

    ge = acc / max_len.astype(_f32)
    g1 = jnp.maximum(
        jnp.dot(ge, wfc1a_ref[...], preferred_element_type=_f32,
                precision=lax.Precision.HIGHEST)
        + focal_ref[...] * wfc1b_ref[...] + bfc1_ref[...],
        0.0)
    o = jnp.dot(g1, wfc2_ref[...], preferred_element_type=_f32,
                precision=lax.Precision.HIGHEST) + bfc2_ref[...]
    out_ref[...] = jax.nn.sigmoid(o)


def _gru_head(gx, WhhT, bihr, bhhr, offs, cnts, ml,
              focal, wfc1a, wfc1b, bfc1r, Wfc2, bfc2r):
    smem = pl.BlockSpec(memory_space=pltpu.SMEM)
    vmem = pl.BlockSpec(memory_space=pltpu.VMEM)
    in_specs = [vmem] * 4 + [smem, smem, smem] + [vmem] * 6
    return pl.pallas_call(
        _gru_body,
        in_specs=in_specs,
        out_specs=pl.BlockSpec(memory_space=pltpu.VMEM),
        out_shape=jax.ShapeDtypeStruct((BGRAPH, 1), _f32),
    )(gx, WhhT, bihr, bhhr, offs, cnts, ml,
      focal, wfc1a, wfc1b, bfc1r, Wfc2, bfc2r)


def kernel(x, edge_index, batch_idx, focal_points, emb, W1, b1, W2, b2,
           W_ih, W_hh, b_ih, b_hh, W_fc1, b_fc1, W_fc2, b_fc2):
    ones16 = jnp.ones((ECHUNK, H), _f32)
    zeros16 = jnp.zeros((ROWS_PER_TILE, H), _f32)
    zeros128 = jnp.zeros((ROWS_PER_TILE, H), _f32)

    erow = edge_index[0]
    ecol = edge_index[1]
    h0, degp = _embed_deg_kernel(x, ecol, emb, ones16, zeros16)
    y1, dis = _proj1(h0, W1, degp)
    p1 = _msg_kernel(erow, ecol, y1, zeros128)
    y2 = _finish_proj(p1, y1, dis, b1.reshape(1, H), W2, post_scale=True)
    p2 = _msg_kernel(erow, ecol, y2, zeros128)
    gx = _finish_proj(p2, y2, dis, b2.reshape(1, H), W_ih.T, post_scale=False)

    ar = jnp.arange(BGRAPH, dtype=batch_idx.dtype)
    offs = jnp.searchsorted(batch_idx, ar, side="left").astype(jnp.int32)
    upper = jnp.searchsorted(batch_idx, ar, side="right").astype(jnp.int32)
    cnts = upper - offs
    ml = jnp.max(cnts).reshape(1)

    out = _gru_head(
        gx, W_hh.T, b_ih.reshape(1, 3 * H), b_hh.reshape(1, 3 * H),
        offs, cnts, ml, focal_points,
        W_fc1[:H], W_fc1[H:H + 1], b_fc1.reshape(1, H), W_fc2,
        b_fc2.reshape(1, 1))
    return out


# speedup vs baseline: 33.8710x; 1.3304x over previous
import functools

import jax
import jax.numpy as jnp
from jax import lax
from jax.experimental import pallas as pl
from jax.experimental.pallas import tpu as pltpu
from jax.experimental.pallas import tpu_sc as plsc

N = 10000
E = 320000
H = 128
BGRAPH = 16
NC = 2
NS = 16
NW = NC * NS
NPAD = 10240
ROWS_PER_TILE = NPAD // NS
ECHUNK = 80
N_ECHUNK = E // ECHUNK
GCHUNK = 80
N_GCHUNK = N // GCHUNK

_f32 = jnp.float32

_sc_mesh = plsc.VectorSubcoreMesh(core_axis_name="c", subcore_axis_name="s")


@functools.partial(
    pl.kernel,
    out_type=[
        jax.ShapeDtypeStruct((N, H), _f32),
        jax.ShapeDtypeStruct((NC, NPAD, H), _f32),
    ],
    mesh=_sc_mesh,
    scratch_types=[
        pltpu.VMEM((GCHUNK,), jnp.int32),
        pltpu.VMEM((GCHUNK, H), _f32),
        pltpu.VMEM((ECHUNK,), jnp.int32),
        pltpu.VMEM((ECHUNK, H), _f32),
        pltpu.VMEM_SHARED((NPAD, H), _f32),
        pltpu.SemaphoreType.DMA,
    ],
)
def _embed_deg_kernel(x_hbm, ecol_hbm, emb_hbm, ones_hbm, zeros_hbm,
                      h0_hbm, degp_hbm,
                      idx_v, rows_v, col_v, ones_v, deg_sh, sem):
    c = lax.axis_index("c")
    s = lax.axis_index("s")
    w = s * NC + c

    pltpu.sync_copy(zeros_hbm,
                    deg_sh.at[pl.ds(s * ROWS_PER_TILE, ROWS_PER_TILE)])
    pltpu.sync_copy(ones_hbm, ones_v)
    plsc.subcore_barrier()

    def gbody(i, carry):
        ch = w + NW * i

        @pl.when(ch < N_GCHUNK)
        def _():
            pltpu.sync_copy(x_hbm.at[pl.ds(ch * GCHUNK, GCHUNK)], idx_v)
            pltpu.async_copy(emb_hbm.at[idx_v], rows_v, sem).wait()
            pltpu.sync_copy(rows_v, h0_hbm.at[pl.ds(ch * GCHUNK, GCHUNK)])

        return carry

    lax.fori_loop(0, (N_GCHUNK + NW - 1) // NW, gbody, 0)

    def dbody(i, carry):
        ch = w + NW * i
        pltpu.sync_copy(ecol_hbm.at[pl.ds(ch * ECHUNK, ECHUNK)], col_v)
        pltpu.sync_copy(ones_v, deg_sh.at[col_v], add=True)
        return carry

    lax.fori_loop(0, N_ECHUNK // NW, dbody, 0)
    plsc.subcore_barrier()

    pltpu.sync_copy(deg_sh.at[pl.ds(s * ROWS_PER_TILE, ROWS_PER_TILE)],
                    degp_hbm.at[c, pl.ds(s * ROWS_PER_TILE, ROWS_PER_TILE)])


@functools.partial(
    pl.kernel,
    out_type=jax.ShapeDtypeStruct((NC, NPAD, H), _f32),
    mesh=_sc_mesh,
    scratch_types=[
        pltpu.VMEM((ECHUNK,), jnp.int32),
        pltpu.VMEM((ECHUNK,), jnp.int32),
        pltpu.VMEM((ECHUNK,), jnp.int32),
        pltpu.VMEM((ECHUNK,), jnp.int32),
        pltpu.VMEM((ECHUNK, H), _f32),
        pltpu.VMEM((ECHUNK, H), _f32),
        pltpu.VMEM_SHARED((NPAD, H), _f32),
        pltpu.SemaphoreType.DMA,
        pltpu.SemaphoreType.DMA,
    ],
)
def _msg_kernel(erow_hbm, ecol_hbm, y_hbm, zeros_hbm, out_hbm,
                row0, row1, col0, col1, msg0, msg1, acc_sh, sem0, sem1):
    c = lax.axis_index("c")
    s = lax.axis_index("s")
    w = s * NC + c
    cpt = N_ECHUNK // NW

    pltpu.sync_copy(zeros_hbm,
                    acc_sh.at[pl.ds(s * ROWS_PER_TILE, ROWS_PER_TILE)])
    plsc.subcore_barrier()

    bufs = ((row0, col0, msg0, sem0), (row1, col1, msg1, sem1))

    def issue(k, rv, cv, mv, sm):
        g = (w + NW * k) * ECHUNK
        pltpu.sync_copy(erow_hbm.at[pl.ds(g, ECHUNK)], rv)
        pltpu.sync_copy(ecol_hbm.at[pl.ds(g, ECHUNK)], cv)
        pltpu.async_copy(y_hbm.at[rv], mv, sm)

    def drain(rv, cv, mv, sm):
        pltpu.make_async_copy(y_hbm.at[rv], mv, sm).wait()
        pltpu.sync_copy(mv, acc_sh.at[cv], add=True)

    issue(0, *bufs[0])

    def body(j, carry):
        for b in (0, 1):
            k = 2 * j + b
            issue(k + 1, *bufs[1 - b])
            drain(*bufs[b])
        return carry

    lax.fori_loop(0, (cpt - 1) // 2, body, 0)
    drain(*bufs[0])
    plsc.subcore_barrier()

    pltpu.sync_copy(acc_sh.at[pl.ds(s * ROWS_PER_TILE, ROWS_PER_TILE)],
                    out_hbm.at[c, pl.ds(s * ROWS_PER_TILE, ROWS_PER_TILE)])


RB = 2000
NRB = N // RB


def _proj1_body(h0_ref, w1_ref, degp_ref, y_ref, dis_ref):
    deg = degp_ref[0, :, 0:1] + degp_ref[1, :, 0:1] + 1.0
    dis = lax.rsqrt(deg)
    xw = jnp.dot(h0_ref[...], w1_ref[...], preferred_element_type=_f32,
                 precision=lax.Precision.HIGHEST)
    y_ref[...] = xw * dis
    dis_ref[...] = dis


def _proj1(h0, W1, degp):
    return pl.pallas_call(
        _proj1_body,
        grid=(NRB,),
        in_specs=[
            pl.BlockSpec((RB, H), lambda i: (i, 0)),
            pl.BlockSpec((H, H), lambda i: (0, 0)),
            pl.BlockSpec((NC, RB, H), lambda i: (0, i, 0)),
        ],
        out_specs=[
            pl.BlockSpec((RB, H), lambda i: (i, 0)),
            pl.BlockSpec((RB, 1), lambda i: (i, 0)),
        ],
        out_shape=[
            jax.ShapeDtypeStruct((N, H), _f32),
            jax.ShapeDtypeStruct((N, 1), _f32),
        ],
    )(h0, W1, degp)


def _finish_proj_body(p_ref, y_ref, dis_ref, b_ref, w_ref, out_ref,
                      *, post_scale):
    dis = dis_ref[...]
    h = jnp.maximum(dis * (p_ref[0] + p_ref[1] + y_ref[...]) + b_ref[...],
                    0.0)
    xw = jnp.dot(h, w_ref[...], preferred_element_type=_f32,
                 precision=lax.Precision.HIGHEST)
    out_ref[...] = xw * dis if post_scale else xw


def _finish_proj(p, y, dis, br, W, post_scale):
    ho = W.shape[1]
    body = functools.partial(_finish_proj_body, post_scale=post_scale)
    return pl.pallas_call(
        body,
        grid=(NRB,),
        in_specs=[
            pl.BlockSpec((NC, RB, H), lambda i: (0, i, 0)),
            pl.BlockSpec((RB, H), lambda i: (i, 0)),
            pl.BlockSpec((RB, 1), lambda i: (i, 0)),
            pl.BlockSpec((1, H), lambda i: (0, 0)),
            pl.BlockSpec((H, ho), lambda i: (0, 0)),
        ],
        out_specs=pl.BlockSpec((RB, ho), lambda i: (i, 0)),
        out_shape=jax.ShapeDtypeStruct((N, ho), _f32),
    )(p, y, dis, br, W)


def _gru_body(gx_ref, whh_ref, bih_ref, bhh_ref, offs_ref, cnts_ref, ml_ref,
              focal_ref, wfc1a_ref, wfc1b_ref, bfc1_ref, wfc2_ref, bfc2_ref,
              out_ref):
    max_len = ml_ref[0]
    bih = bih_ref[...]
    bhh = bhh_ref[...]
    whh = whh_ref[...]

    def step(t, carry):
        hst, acc = carry
        rows = []
        for b in range(BGRAPH):
            valid = t < cnts_ref[b]
            safe = jnp.where(valid, offs_ref[b] + t, 0)
            r = gx_ref[pl.ds(safe, 1), :]
            rows.append(jnp.where(valid, r, 0.0))
        gx = jnp.concatenate(rows, axis=0) + bih
        gh = jnp.dot(hst, whh, preferred_element_type=_f32,
                     precision=lax.Precision.HIGHEST) + bhh
        r_ = jax.nn.sigmoid(gx[:, :H] + gh[:, :H])
        z = jax.nn.sigmoid(gx[:, H:2 * H] + gh[:, H:2 * H])
        n_ = jnp.tanh(gx[:, 2 * H:] + r_ * gh[:, 2 * H:])
        hst = (1.0 - z) * n_ + z * hst
        return hst, acc + hst

    init = (jnp.zeros((BGRAPH, H), _f32), jnp.zeros((BGRAPH, H), _f32))
    _, acc = lax.fori_loop(0, max_len, step, init)

    ge = acc / max_len.astype(_f32)
    g1 = jnp.maximum(
        jnp.dot(ge, wfc1a_ref[...], preferred_element_type=_f32,
                precision=lax.Precision.HIGHEST)
        + focal_ref[...] * wfc1b_ref[...] + bfc1_ref[...],
        0.0)
    o = jnp.dot(g1, wfc2_ref[...], preferred_element_type=_f32,
                precision=lax.Precision.HIGHEST) + bfc2_ref[...]
    out_ref[...] = jax.nn.sigmoid(o)


def _gru_head(gx, WhhT, bihr, bhhr, offs, cnts, ml,
              focal, wfc1a, wfc1b, bfc1r, Wfc2, bfc2r):
    smem = pl.BlockSpec(memory_space=pltpu.SMEM)
    vmem = pl.BlockSpec(memory_space=pltpu.VMEM)
    in_specs = [vmem] * 4 + [smem, smem, smem] + [vmem] * 6
    return pl.pallas_call(
        _gru_body,
        in_specs=in_specs,
        out_specs=pl.BlockSpec(memory_space=pltpu.VMEM),
        out_shape=jax.ShapeDtypeStruct((BGRAPH, 1), _f32),
    )(gx, WhhT, bihr, bhhr, offs, cnts, ml,
      focal, wfc1a, wfc1b, bfc1r, Wfc2, bfc2r)


def kernel(x, edge_index, batch_idx, focal_points, emb, W1, b1, W2, b2,
           W_ih, W_hh, b_ih, b_hh, W_fc1, b_fc1, W_fc2, b_fc2):
    ones16 = jnp.ones((ECHUNK, H), _f32)
    zeros16 = jnp.zeros((ROWS_PER_TILE, H), _f32)
    zeros128 = jnp.zeros((ROWS_PER_TILE, H), _f32)

    erow = edge_index[0]
    ecol = edge_index[1]
    h0, degp = _embed_deg_kernel(x, ecol, emb, ones16, zeros16)
    y1, dis = _proj1(h0, W1, degp)
    p1 = _msg_kernel(erow, ecol, y1, zeros128)
    y2 = _finish_proj(p1, y1, dis, b1.reshape(1, H), W2, post_scale=True)
    p2 = _msg_kernel(erow, ecol, y2, zeros128)
    gx = _finish_proj(p2, y2, dis, b2.reshape(1, H), W_ih.T, post_scale=False)

    ar = jnp.arange(BGRAPH, dtype=batch_idx.dtype)
    offs = jnp.searchsorted(batch_idx, ar, side="left").astype(jnp.int32)
    upper = jnp.searchsorted(batch_idx, ar, side="right").astype(jnp.int32)
    cnts = upper - offs
    ml = jnp.max(cnts).reshape(1)

    out = _gru_head(
        gx, W_hh.T, b_ih.reshape(1, 3 * H), b_hh.reshape(1, 3 * H),
        offs, cnts, ml, focal_points,
        W_fc1[:H], W_fc1[H:H + 1], b_fc1.reshape(1, H), W_fc2,
        b_fc2.reshape(1, 1))
    return out
